# baseline (speedup 1.0000x reference)
import functools

import jax
import jax.numpy as jnp
from jax import lax
from jax.experimental import pallas as pl
from jax.experimental.pallas import tpu as pltpu
from jax.experimental.pallas import tpu_sc as plsc

N_NODES = 10000
N_EDGES = 320000
D = 128

NS = 16

NC = 2

CHUNK = 125
NCH = 80
G = 4
NSL = NCH // G
TOT_SLABS = NC * NS * NSL
ACC_ROWS = 10112
RPT = ACC_ROWS // NS

_sc_mesh = plsc.VectorSubcoreMesh(core_axis_name="c", subcore_axis_name="s")


@functools.partial(
    pl.kernel,
    mesh=_sc_mesh,
    out_type=[
        jax.ShapeDtypeStruct((ACC_ROWS, D), jnp.float32),
        jax.ShapeDtypeStruct((ACC_ROWS, D), jnp.float32),
    ],
    scratch_types=[
        pltpu.VMEM((2, G, CHUNK), jnp.int32),
        pltpu.VMEM((2, G, CHUNK), jnp.int32),
        pltpu.VMEM((2, CHUNK, D), jnp.float32),
        pltpu.VMEM_SHARED((ACC_ROWS, D), jnp.float32),
        pltpu.SemaphoreType.DMA,
        pltpu.SemaphoreType.DMA,
        pltpu.SemaphoreType.DMA,
        pltpu.SemaphoreType.DMA,
    ],
)
def _sc_agg(h_hbm, src_hbm, dst_hbm, zeros_hbm, p0_hbm, p1_hbm,
            src_sl, dst_sl, rows_v, acc, rs0, rs1, is0, is1):
    c = lax.axis_index("c")
    s = lax.axis_index("s")
    rsem = (rs0, rs1)
    isem = (is0, is1)
    sbase = (c * NS + s) * NSL

    def idx_start(t, p):
        pltpu.make_async_copy(src_hbm.at[sbase + t], src_sl.at[p],
                              isem[p]).start()
        pltpu.make_async_copy(dst_hbm.at[sbase + t], dst_sl.at[p],
                              isem[p]).start()

    def idx_wait(t, p):
        pltpu.make_async_copy(src_hbm.at[sbase + t], src_sl.at[p],
                              isem[p]).wait()
        pltpu.make_async_copy(dst_hbm.at[sbase + t], dst_sl.at[p],
                              isem[p]).wait()

    def row_start(p, g, rb):
        pltpu.make_async_copy(h_hbm.at[src_sl.at[p, g]], rows_v.at[rb],
                              rsem[rb]).start()

    def row_wait(p, g, rb):
        pltpu.make_async_copy(h_hbm.at[src_sl.at[p, g]], rows_v.at[rb],
                              rsem[rb]).wait()

    idx_start(0, 0)
    idx_start(1, 1)
    pltpu.sync_copy(zeros_hbm.at[pl.ds(s * RPT, RPT)],
                    acc.at[pl.ds(s * RPT, RPT)])
    plsc.subcore_barrier()

    idx_wait(0, 0)
    row_start(0, 0, 0)
    row_start(0, 1, 1)

    def outer(k, carry):
        for p in (0, 1):
            t = 2 * k + p
            for g in range(G):
                rb = g % 2
                row_wait(p, g, rb)
                pltpu.sync_copy(rows_v.at[rb], acc.at[dst_sl.at[p, g]],
                                add=True)
                if g < G - 2:
                    row_start(p, g + 2, rb)
                else:
                    @pl.when(t + 1 < NSL)
                    def _(p=p, g=g, rb=rb, t=t):
                        if g == G - 2:
                            idx_wait(t + 1, 1 - p)
                        row_start(1 - p, g + 2 - G, rb)

            @pl.when(t + 2 < NSL)
            def _(p=p, t=t):
                idx_start(t + 2, p)
        return carry

    lax.fori_loop(0, NSL // 2, outer, 0)
    plsc.subcore_barrier()

    @pl.when(c == 0)
    def _():
        pltpu.sync_copy(acc.at[pl.ds(s * RPT, RPT)],
                        p0_hbm.at[pl.ds(s * RPT, RPT)])

    @pl.when(c == 1)
    def _():
        pltpu.sync_copy(acc.at[pl.ds(s * RPT, RPT)],
                        p1_hbm.at[pl.ds(s * RPT, RPT)])


def _mm_body(p0_ref, p1_ref, x_ref, wt_ref, b_ref, o_ref):
    agg = p0_ref[...] + p1_ref[...] + x_ref[...]
    y = jnp.dot(agg, wt_ref[...], preferred_element_type=jnp.float32)
    o_ref[...] = jnp.maximum(y + b_ref[...], 0.0)


_BM = 1000


def _tc_layer(p0, p1, xin, wt, b):
    return pl.pallas_call(
        _mm_body,
        grid=(N_NODES // _BM,),
        in_specs=[
            pl.BlockSpec((_BM, D), lambda i: (i, 0)),
            pl.BlockSpec((_BM, D), lambda i: (i, 0)),
            pl.BlockSpec((_BM, D), lambda i: (i, 0)),
            pl.BlockSpec((D, D), lambda i: (0, 0)),
            pl.BlockSpec((1, D), lambda i: (0, 0)),
        ],
        out_specs=pl.BlockSpec((_BM, D), lambda i: (i, 0)),
        out_shape=jax.ShapeDtypeStruct((N_NODES, D), jnp.float32),
    )(p0, p1, xin, wt, b)


def kernel(x, edge_index, W1, b1, W2, b2):
    src = edge_index[0].astype(jnp.int32)
    dst = edge_index[1].astype(jnp.int32)
    src_p = src.reshape(TOT_SLABS, G, CHUNK)
    dst_p = dst.reshape(TOT_SLABS, G, CHUNK)
    zeros = jnp.zeros((ACC_ROWS, D), jnp.float32)
    wt1 = W1.T
    wt2 = W2.T
    b1r = b1.reshape(1, D)
    b2r = b2.reshape(1, D)

    p0, p1 = _sc_agg(x, src_p, dst_p, zeros)
    h1 = _tc_layer(p0, p1, x, wt1, b1r)
    q0, q1 = _sc_agg(h1, src_p, dst_p, zeros)
    h2 = _tc_layer(q0, q1, h1, wt2, b2r)
    return h2


# speedup vs baseline: 1.0443x; 1.0443x over previous
import functools

import jax
import jax.numpy as jnp
from jax import lax
from jax.experimental import pallas as pl
from jax.experimental.pallas import tpu as pltpu
from jax.experimental.pallas import tpu_sc as plsc

N_NODES = 10000
N_EDGES = 320000
D = 128

NS = 16

NC = 2

CHUNK = 125
NCH = 80
G = 4
NSL = NCH // G
TOT_SLABS = NC * NS * NSL
ACC_ROWS = 10112
RPT = ACC_ROWS // NS

_sc_mesh = plsc.VectorSubcoreMesh(core_axis_name="c", subcore_axis_name="s")


@functools.partial(
    pl.kernel,
    mesh=_sc_mesh,
    out_type=[
        jax.ShapeDtypeStruct((ACC_ROWS, D), jnp.float32),
        jax.ShapeDtypeStruct((ACC_ROWS, D), jnp.float32),
    ],
    scratch_types=[
        pltpu.VMEM((2, G, CHUNK), jnp.int32),
        pltpu.VMEM((2, G, CHUNK), jnp.int32),
        pltpu.VMEM((2, CHUNK, D), jnp.float32),
        pltpu.VMEM_SHARED((ACC_ROWS, D), jnp.float32),
        pltpu.SemaphoreType.DMA,
        pltpu.SemaphoreType.DMA,
        pltpu.SemaphoreType.DMA,
        pltpu.SemaphoreType.DMA,
    ],
)
def _sc_agg(h_hbm, ei_hbm, zeros_hbm, p0_hbm, p1_hbm,
            src_sl, dst_sl, rows_v, acc, rs0, rs1, is0, is1):
    c = lax.axis_index("c")
    s = lax.axis_index("s")
    rsem = (rs0, rs1)
    isem = (is0, is1)
    sbase = (c * NS + s) * NSL

    def idx_start(t, p):
        pltpu.make_async_copy(ei_hbm.at[0, sbase + t], src_sl.at[p],
                              isem[p]).start()
        pltpu.make_async_copy(ei_hbm.at[1, sbase + t], dst_sl.at[p],
                              isem[p]).start()

    def idx_wait(t, p):
        pltpu.make_async_copy(ei_hbm.at[0, sbase + t], src_sl.at[p],
                              isem[p]).wait()
        pltpu.make_async_copy(ei_hbm.at[1, sbase + t], dst_sl.at[p],
                              isem[p]).wait()

    def row_start(p, g, rb):
        pltpu.make_async_copy(h_hbm.at[src_sl.at[p, g]], rows_v.at[rb],
                              rsem[rb]).start()

    def row_wait(p, g, rb):
        pltpu.make_async_copy(h_hbm.at[src_sl.at[p, g]], rows_v.at[rb],
                              rsem[rb]).wait()

    idx_start(0, 0)
    idx_start(1, 1)
    pltpu.sync_copy(zeros_hbm.at[pl.ds(s * RPT, RPT)],
                    acc.at[pl.ds(s * RPT, RPT)])
    plsc.subcore_barrier()

    idx_wait(0, 0)
    row_start(0, 0, 0)
    row_start(0, 1, 1)

    def outer(k, carry):
        for p in (0, 1):
            t = 2 * k + p
            for g in range(G):
                rb = g % 2
                row_wait(p, g, rb)
                pltpu.sync_copy(rows_v.at[rb], acc.at[dst_sl.at[p, g]],
                                add=True)
                if g < G - 2:
                    row_start(p, g + 2, rb)
                else:
                    @pl.when(t + 1 < NSL)
                    def _(p=p, g=g, rb=rb, t=t):
                        if g == G - 2:
                            idx_wait(t + 1, 1 - p)
                        row_start(1 - p, g + 2 - G, rb)

            @pl.when(t + 2 < NSL)
            def _(p=p, t=t):
                idx_start(t + 2, p)
        return carry

    lax.fori_loop(0, NSL // 2, outer, 0)
    plsc.subcore_barrier()

    @pl.when(c == 0)
    def _():
        pltpu.sync_copy(acc.at[pl.ds(s * RPT, RPT)],
                        p0_hbm.at[pl.ds(s * RPT, RPT)])

    @pl.when(c == 1)
    def _():
        pltpu.sync_copy(acc.at[pl.ds(s * RPT, RPT)],
                        p1_hbm.at[pl.ds(s * RPT, RPT)])


def _mm_body(p0_ref, p1_ref, x_ref, wt_ref, b_ref, o_ref):
    agg = p0_ref[...] + p1_ref[...] + x_ref[...]
    y = jnp.dot(agg, wt_ref[...], preferred_element_type=jnp.float32)
    o_ref[...] = jnp.maximum(y + b_ref[...], 0.0)


_BM = 1000


def _tc_layer(p0, p1, xin, wt, b):
    return pl.pallas_call(
        _mm_body,
        grid=(N_NODES // _BM,),
        in_specs=[
            pl.BlockSpec((_BM, D), lambda i: (i, 0)),
            pl.BlockSpec((_BM, D), lambda i: (i, 0)),
            pl.BlockSpec((_BM, D), lambda i: (i, 0)),
            pl.BlockSpec((D, D), lambda i: (0, 0)),
            pl.BlockSpec((1, D), lambda i: (0, 0)),
        ],
        out_specs=pl.BlockSpec((_BM, D), lambda i: (i, 0)),
        out_shape=jax.ShapeDtypeStruct((N_NODES, D), jnp.float32),
    )(p0, p1, xin, wt, b)


def kernel(x, edge_index, W1, b1, W2, b2):
    ei = edge_index.astype(jnp.int32).reshape(2, TOT_SLABS, G, CHUNK)
    zeros = jnp.zeros((ACC_ROWS, D), jnp.float32)
    wt1 = W1.T
    wt2 = W2.T
    b1r = b1.reshape(1, D)
    b2r = b2.reshape(1, D)

    p0, p1 = _sc_agg(x, ei, zeros)
    h1 = _tc_layer(p0, p1, x, wt1, b1r)
    q0, q1 = _sc_agg(h1, ei, zeros)
    h2 = _tc_layer(q0, q1, h1, wt2, b2r)
    return h2


# speedup vs baseline: 1.0480x; 1.0036x over previous
import functools

import jax
import jax.numpy as jnp
from jax import lax
from jax.experimental import pallas as pl
from jax.experimental.pallas import tpu as pltpu
from jax.experimental.pallas import tpu_sc as plsc

N_NODES = 10000
N_EDGES = 320000
D = 128

NS = 16

NC = 2

CHUNK = 125
NCH = 80
G = 4
NSL = NCH // G
TOT_SLABS = NC * NS * NSL
ACC_ROWS = 10112
RPT = ACC_ROWS // NS

_sc_mesh = plsc.VectorSubcoreMesh(core_axis_name="c", subcore_axis_name="s")


@functools.partial(
    pl.kernel,
    mesh=_sc_mesh,
    out_type=[
        jax.ShapeDtypeStruct((ACC_ROWS, D), jnp.float32),
        jax.ShapeDtypeStruct((ACC_ROWS, D), jnp.float32),
    ],
    scratch_types=[
        pltpu.VMEM((2, G, CHUNK), jnp.int32),
        pltpu.VMEM((2, G, CHUNK), jnp.int32),
        pltpu.VMEM((2, CHUNK, D), jnp.float32),
        pltpu.VMEM_SHARED((ACC_ROWS, D), jnp.float32),
        pltpu.SemaphoreType.DMA,
        pltpu.SemaphoreType.DMA,
        pltpu.SemaphoreType.DMA,
        pltpu.SemaphoreType.DMA,
    ],
)
def _sc_agg(h_hbm, ei_hbm, p0_hbm, p1_hbm,
            src_sl, dst_sl, rows_v, acc, rs0, rs1, is0, is1):
    c = lax.axis_index("c")
    s = lax.axis_index("s")
    rsem = (rs0, rs1)
    isem = (is0, is1)
    sbase = (c * NS + s) * NSL

    def idx_start(t, p):
        pltpu.make_async_copy(ei_hbm.at[0, sbase + t], src_sl.at[p],
                              isem[p]).start()
        pltpu.make_async_copy(ei_hbm.at[1, sbase + t], dst_sl.at[p],
                              isem[p]).start()

    def idx_wait(t, p):
        pltpu.make_async_copy(ei_hbm.at[0, sbase + t], src_sl.at[p],
                              isem[p]).wait()
        pltpu.make_async_copy(ei_hbm.at[1, sbase + t], dst_sl.at[p],
                              isem[p]).wait()

    def row_start(p, g, rb):
        pltpu.make_async_copy(h_hbm.at[src_sl.at[p, g]], rows_v.at[rb],
                              rsem[rb]).start()

    def row_wait(p, g, rb):
        pltpu.make_async_copy(h_hbm.at[src_sl.at[p, g]], rows_v.at[rb],
                              rsem[rb]).wait()

    idx_start(0, 0)
    idx_start(1, 1)

    def zstore(i, carry):
        rows_v[0, i // 8, pl.ds((i % 8) * 16, 16)] = jnp.zeros(
            (16,), jnp.float32)
        return carry

    lax.fori_loop(0, 120 * 8, zstore, 0)
    for k in range(5):
        pltpu.sync_copy(rows_v.at[0, pl.ds(0, 120)],
                        acc.at[pl.ds(s * RPT + k * 120, 120)])
    pltpu.sync_copy(rows_v.at[0, pl.ds(0, 32)],
                    acc.at[pl.ds(s * RPT + 600, 32)])
    plsc.subcore_barrier()

    idx_wait(0, 0)
    row_start(0, 0, 0)
    row_start(0, 1, 1)

    def outer(k, carry):
        for p in (0, 1):
            t = 2 * k + p
            for g in range(G):
                rb = g % 2
                row_wait(p, g, rb)
                pltpu.sync_copy(rows_v.at[rb], acc.at[dst_sl.at[p, g]],
                                add=True)
                if g < G - 2:
                    row_start(p, g + 2, rb)
                else:
                    @pl.when(t + 1 < NSL)
                    def _(p=p, g=g, rb=rb, t=t):
                        if g == G - 2:
                            idx_wait(t + 1, 1 - p)
                        row_start(1 - p, g + 2 - G, rb)

            @pl.when(t + 2 < NSL)
            def _(p=p, t=t):
                idx_start(t + 2, p)
        return carry

    lax.fori_loop(0, NSL // 2, outer, 0)
    plsc.subcore_barrier()

    @pl.when(c == 0)
    def _():
        pltpu.sync_copy(acc.at[pl.ds(s * RPT, RPT)],
                        p0_hbm.at[pl.ds(s * RPT, RPT)])

    @pl.when(c == 1)
    def _():
        pltpu.sync_copy(acc.at[pl.ds(s * RPT, RPT)],
                        p1_hbm.at[pl.ds(s * RPT, RPT)])


def _mm_body(p0_ref, p1_ref, x_ref, wt_ref, b_ref, o_ref):
    agg = p0_ref[...] + p1_ref[...] + x_ref[...]
    y = jnp.dot(agg, wt_ref[...], preferred_element_type=jnp.float32)
    o_ref[...] = jnp.maximum(y + b_ref[...], 0.0)


_BM = 1000


def _tc_layer(p0, p1, xin, wt, b):
    return pl.pallas_call(
        _mm_body,
        grid=(N_NODES // _BM,),
        in_specs=[
            pl.BlockSpec((_BM, D), lambda i: (i, 0)),
            pl.BlockSpec((_BM, D), lambda i: (i, 0)),
            pl.BlockSpec((_BM, D), lambda i: (i, 0)),
            pl.BlockSpec((D, D), lambda i: (0, 0)),
            pl.BlockSpec((1, D), lambda i: (0, 0)),
        ],
        out_specs=pl.BlockSpec((_BM, D), lambda i: (i, 0)),
        out_shape=jax.ShapeDtypeStruct((N_NODES, D), jnp.float32),
    )(p0, p1, xin, wt, b)


def kernel(x, edge_index, W1, b1, W2, b2):
    ei = edge_index.astype(jnp.int32).reshape(2, TOT_SLABS, G, CHUNK)
    wt1 = W1.T
    wt2 = W2.T
    b1r = b1.reshape(1, D)
    b2r = b2.reshape(1, D)

    p0, p1 = _sc_agg(x, ei)
    h1 = _tc_layer(p0, p1, x, wt1, b1r)
    q0, q1 = _sc_agg(h1, ei)
    h2 = _tc_layer(q0, q1, h1, wt2, b2r)
    return h2


# speedup vs baseline: 1.0685x; 1.0196x over previous
import functools

import jax
import jax.numpy as jnp
from jax import lax
from jax.experimental import pallas as pl
from jax.experimental.pallas import tpu as pltpu
from jax.experimental.pallas import tpu_sc as plsc

N_NODES = 10000
N_EDGES = 320000
D = 128

NS = 16

NC = 2

CHUNK = 125
NCH = 80
G = 4
NSL = NCH // G
TOT_SLABS = NC * NS * NSL
ACC_ROWS = 10112
RPT = ACC_ROWS // NS

_sc_mesh = plsc.VectorSubcoreMesh(core_axis_name="c", subcore_axis_name="s")


@functools.partial(
    pl.kernel,
    mesh=_sc_mesh,
    out_type=[
        jax.ShapeDtypeStruct((ACC_ROWS, D), jnp.float32),
        jax.ShapeDtypeStruct((ACC_ROWS, D), jnp.float32),
    ],
    scratch_types=[
        pltpu.VMEM((2, G, CHUNK), jnp.int32),
        pltpu.VMEM((2, G, CHUNK), jnp.int32),
        pltpu.VMEM((2, CHUNK, D), jnp.float32),
        pltpu.VMEM_SHARED((ACC_ROWS, D), jnp.float32),
        pltpu.SemaphoreType.DMA,
        pltpu.SemaphoreType.DMA,
        pltpu.SemaphoreType.DMA,
        pltpu.SemaphoreType.DMA,
    ],
)
def _sc_agg(h_hbm, ei_hbm, p0_hbm, p1_hbm,
            src_sl, dst_sl, rows_v, acc, rs0, rs1, is0, is1):
    c = lax.axis_index("c")
    s = lax.axis_index("s")
    rsem = (rs0, rs1)
    isem = (is0, is1)
    sbase = (c * NS + s) * NSL

    def idx_start(t, p):
        pltpu.make_async_copy(ei_hbm.at[0, sbase + t], src_sl.at[p],
                              isem[p]).start()
        pltpu.make_async_copy(ei_hbm.at[1, sbase + t], dst_sl.at[p],
                              isem[p]).start()

    def idx_wait(t, p):
        pltpu.make_async_copy(ei_hbm.at[0, sbase + t], src_sl.at[p],
                              isem[p]).wait()
        pltpu.make_async_copy(ei_hbm.at[1, sbase + t], dst_sl.at[p],
                              isem[p]).wait()

    def row_start(p, g, rb):
        pltpu.make_async_copy(h_hbm.at[src_sl.at[p, g]], rows_v.at[rb],
                              rsem[rb]).start()

    def row_wait(p, g, rb):
        pltpu.make_async_copy(h_hbm.at[src_sl.at[p, g]], rows_v.at[rb],
                              rsem[rb]).wait()

    idx_start(0, 0)
    idx_start(1, 1)

    def zstore(i, carry):
        rows_v[0, i // 8, pl.ds((i % 8) * 16, 16)] = jnp.zeros(
            (16,), jnp.float32)
        return carry

    lax.fori_loop(0, 120 * 8, zstore, 0)
    for k in range(5):
        pltpu.sync_copy(rows_v.at[0, pl.ds(0, 120)],
                        acc.at[pl.ds(s * RPT + k * 120, 120)])
    pltpu.sync_copy(rows_v.at[0, pl.ds(0, 32)],
                    acc.at[pl.ds(s * RPT + 600, 32)])
    plsc.subcore_barrier()

    idx_wait(0, 0)
    row_start(0, 0, 0)
    row_start(0, 1, 1)

    def outer(k, carry):
        for p in (0, 1):
            t = 2 * k + p
            for g in range(G):
                rb = g % 2
                row_wait(p, g, rb)
                pltpu.sync_copy(rows_v.at[rb], acc.at[dst_sl.at[p, g]],
                                add=True)
                if g < G - 2:
                    row_start(p, g + 2, rb)
                else:
                    @pl.when(t + 1 < NSL)
                    def _(p=p, g=g, rb=rb, t=t):
                        if g == G - 2:
                            idx_wait(t + 1, 1 - p)
                        row_start(1 - p, g + 2 - G, rb)

            @pl.when(t + 2 < NSL)
            def _(p=p, t=t):
                idx_start(t + 2, p)
        return carry

    lax.fori_loop(0, NSL // 2, outer, 0)
    plsc.subcore_barrier()

    @pl.when(c == 0)
    def _():
        pltpu.sync_copy(acc.at[pl.ds(s * RPT, RPT)],
                        p0_hbm.at[pl.ds(s * RPT, RPT)])

    @pl.when(c == 1)
    def _():
        pltpu.sync_copy(acc.at[pl.ds(s * RPT, RPT)],
                        p1_hbm.at[pl.ds(s * RPT, RPT)])


def _mm_body(p0_ref, p1_ref, x_ref, wt_ref, b_ref, o_ref):
    agg = p0_ref[...] + p1_ref[...] + x_ref[...]
    y = jnp.dot(agg, wt_ref[...], preferred_element_type=jnp.float32)
    o_ref[...] = jnp.maximum(y + b_ref[...], 0.0)


_BM = 2000


def _tc_layer(p0, p1, xin, wt, b):
    return pl.pallas_call(
        _mm_body,
        grid=(N_NODES // _BM,),
        in_specs=[
            pl.BlockSpec((_BM, D), lambda i: (i, 0)),
            pl.BlockSpec((_BM, D), lambda i: (i, 0)),
            pl.BlockSpec((_BM, D), lambda i: (i, 0)),
            pl.BlockSpec((D, D), lambda i: (0, 0)),
            pl.BlockSpec((1, D), lambda i: (0, 0)),
        ],
        out_specs=pl.BlockSpec((_BM, D), lambda i: (i, 0)),
        out_shape=jax.ShapeDtypeStruct((N_NODES, D), jnp.float32),
    )(p0, p1, xin, wt, b)


def kernel(x, edge_index, W1, b1, W2, b2):
    ei = edge_index.astype(jnp.int32).reshape(2, TOT_SLABS, G, CHUNK)
    wt1 = W1.T
    wt2 = W2.T
    b1r = b1.reshape(1, D)
    b2r = b2.reshape(1, D)

    p0, p1 = _sc_agg(x, ei)
    h1 = _tc_layer(p0, p1, x, wt1, b1r)
    q0, q1 = _sc_agg(h1, ei)
    h2 = _tc_layer(q0, q1, h1, wt2, b2r)
    return h2
